# Initial kernel scaffold; baseline (speedup 1.0000x reference)
import jax
import jax.numpy as jnp
from jax.experimental import pallas as pl


def kernel(z_feats, codebook):
    raise NotImplementedError("write your pallas kernel here")


# speedup vs baseline: 1.3489x; 1.3489x over previous
import functools

import jax
import jax.numpy as jnp
from jax import lax
from jax.experimental import pallas as pl
from jax.experimental.pallas import tpu as pltpu
from jax.experimental.pallas import tpu_sc as plsc

N = 65536
D = 64
K = 8192
BN = 256
NB = N // BN

NC = 2
NS = 16
NW = NC * NS
CHUNK = 128
CHUNKS_PER_W = N // (NW * CHUNK)
ROWS_PER_W = CHUNK * CHUNKS_PER_W


def _argmin_body(z_ref, cb_ref, idx_ref, idxf_ref, part_ref):
    z = z_ref[...]
    cb = cb_ref[...]
    dot = lax.dot_general(z, cb, (((1,), (1,)), ((), ())),
                          preferred_element_type=jnp.float32)
    z2 = jnp.sum(z * z, axis=1, keepdims=True)
    c2 = jnp.sum(cb * cb, axis=1)
    d2 = (z2 + c2[None, :]) - 2.0 * dot
    d2 = jnp.maximum(d2, 0.0)
    bm = jnp.min(d2, axis=1, keepdims=True)
    ii = lax.broadcasted_iota(jnp.int32, d2.shape, 1)
    loc = jnp.min(jnp.where(d2 == bm, ii, jnp.int32(K)), axis=1,
                  keepdims=True)
    idx_ref[...] = loc
    idxf_ref[...] = loc.astype(jnp.float32)
    part_ref[...] = jnp.broadcast_to(jnp.sum(bm), (1, 1, 128))


_distance_argmin = pl.pallas_call(
    _argmin_body,
    grid=(NB,),
    in_specs=[
        pl.BlockSpec((BN, D), lambda i: (i, 0)),
        pl.BlockSpec((K, D), lambda i: (0, 0)),
    ],
    out_specs=[
        pl.BlockSpec((BN, 1), lambda i: (i, 0)),
        pl.BlockSpec((BN, 1), lambda i: (i, 0)),
        pl.BlockSpec((1, 1, 128), lambda i: (i, 0, 0)),
    ],
    out_shape=[
        jax.ShapeDtypeStruct((N, 1), jnp.int32),
        jax.ShapeDtypeStruct((N, 1), jnp.float32),
        jax.ShapeDtypeStruct((NB, 1, 128), jnp.float32),
    ],
)


DPAD = 128


@functools.cache
def _make_sc_gather():
    @functools.partial(
        pl.kernel,
        out_type=jax.ShapeDtypeStruct((N, DPAD), jnp.float32),
        mesh=plsc.VectorSubcoreMesh(core_axis_name="c", subcore_axis_name="s"),
        scratch_types=[
            pltpu.VMEM((CHUNKS_PER_W, CHUNK), jnp.int32),
            pltpu.VMEM((CHUNK, DPAD), jnp.float32),
            pltpu.SemaphoreType.DMA,
        ],
    )
    def _sc_gather(idx_hbm, table_hbm, out_hbm, idx_v, rows_v, sem):
        wid = lax.axis_index("s") * NC + lax.axis_index("c")
        pltpu.sync_copy(
            idx_hbm.at[pl.ds(wid * CHUNKS_PER_W, CHUNKS_PER_W)], idx_v)
        base = wid * ROWS_PER_W
        for c in range(CHUNKS_PER_W):
            pltpu.async_copy(table_hbm.at[idx_v.at[c]], rows_v, sem).wait()
            pltpu.sync_copy(
                rows_v, out_hbm.at[pl.ds(base + c * CHUNK, CHUNK)])

    return _sc_gather


def kernel(z_feats, codebook):
    idx_i32, idx_f, parts = _distance_argmin(z_feats, codebook)
    loss = jnp.sum(parts[:, 0, 0]) / jnp.float32(N * D)
    idx2d = idx_i32.reshape(NW * CHUNKS_PER_W, CHUNK)
    cb_pad = jnp.pad(codebook, ((0, 0), (0, DPAD - D)))
    quantized = _make_sc_gather()(idx2d, cb_pad)[:, :D]
    return quantized, loss, loss, idx_f


# speedup vs baseline: 1.5604x; 1.1568x over previous
import functools

import jax
import jax.numpy as jnp
from jax import lax
from jax.experimental import pallas as pl
from jax.experimental.pallas import tpu as pltpu
from jax.experimental.pallas import tpu_sc as plsc

N = 65536
D = 64
K = 8192
BN = 256
NB = N // BN

NC = 2
NS = 16
NW = NC * NS
CHUNK = 128
CHUNKS_PER_W = N // (NW * CHUNK)
ROWS_PER_W = CHUNK * CHUNKS_PER_W


def _argmin_body(z_ref, cb_ref, idx_ref, idxf_ref, part_ref):
    z = z_ref[...]
    cb = cb_ref[...]
    dot2 = lax.dot_general(z, cb, (((1,), (1,)), ((), ())),
                           preferred_element_type=jnp.float32)
    z2 = jnp.sum(z * z, axis=1, keepdims=True) * 0.25
    c2 = jnp.sum(cb * cb, axis=1)
    d2 = (z2 + c2[None, :]) - dot2
    bm = jnp.min(d2, axis=1, keepdims=True)
    ii = lax.broadcasted_iota(jnp.int32, d2.shape, 1)
    loc = jnp.min(jnp.where(d2 == bm, ii, jnp.int32(K)), axis=1,
                  keepdims=True)
    idx_ref[...] = loc
    idxf_ref[...] = loc.astype(jnp.float32)
    part_ref[...] = jnp.broadcast_to(jnp.sum(jnp.maximum(bm, 0.0)),
                                     (1, 1, 128))


_distance_argmin = pl.pallas_call(
    _argmin_body,
    grid=(NB,),
    in_specs=[
        pl.BlockSpec((BN, D), lambda i: (i, 0)),
        pl.BlockSpec((K, D), lambda i: (0, 0)),
    ],
    out_specs=[
        pl.BlockSpec((BN, 1), lambda i: (i, 0)),
        pl.BlockSpec((BN, 1), lambda i: (i, 0)),
        pl.BlockSpec((1, 1, 128), lambda i: (i, 0, 0)),
    ],
    out_shape=[
        jax.ShapeDtypeStruct((N, 1), jnp.int32),
        jax.ShapeDtypeStruct((N, 1), jnp.float32),
        jax.ShapeDtypeStruct((NB, 1, 128), jnp.float32),
    ],
)


DPAD = 128


@functools.cache
def _make_sc_gather():
    @functools.partial(
        pl.kernel,
        out_type=jax.ShapeDtypeStruct((N, DPAD), jnp.float32),
        mesh=plsc.VectorSubcoreMesh(core_axis_name="c", subcore_axis_name="s"),
        scratch_types=[
            pltpu.VMEM((CHUNKS_PER_W, CHUNK), jnp.int32),
            pltpu.VMEM((CHUNK, DPAD), jnp.float32),
            pltpu.SemaphoreType.DMA,
        ],
    )
    def _sc_gather(idx_hbm, table_hbm, out_hbm, idx_v, rows_v, sem):
        wid = lax.axis_index("s") * NC + lax.axis_index("c")
        pltpu.sync_copy(
            idx_hbm.at[pl.ds(wid * CHUNKS_PER_W, CHUNKS_PER_W)], idx_v)
        base = wid * ROWS_PER_W
        for c in range(CHUNKS_PER_W):
            pltpu.async_copy(table_hbm.at[idx_v.at[c]], rows_v, sem).wait()
            pltpu.sync_copy(
                rows_v, out_hbm.at[pl.ds(base + c * CHUNK, CHUNK)])

    return _sc_gather


def kernel(z_feats, codebook):
    idx_i32, idx_f, parts = _distance_argmin(z_feats + z_feats, codebook)
    loss = jnp.sum(parts[:, 0, 0]) / jnp.float32(N * D)
    idx2d = idx_i32.reshape(NW * CHUNKS_PER_W, CHUNK)
    cb_pad = jnp.pad(codebook, ((0, 0), (0, DPAD - D)))
    quantized = _make_sc_gather()(idx2d, cb_pad)[:, :D]
    return quantized, loss, loss, idx_f


# speedup vs baseline: 1.7101x; 1.0960x over previous
import functools

import jax
import jax.numpy as jnp
from jax import lax
from jax.experimental import pallas as pl
from jax.experimental.pallas import tpu as pltpu
from jax.experimental.pallas import tpu_sc as plsc

N = 65536
D = 64
K = 8192
BN = 1024
NB = N // BN

NC = 2
NS = 16
NW = NC * NS
CHUNK = 128
CHUNKS_PER_W = N // (NW * CHUNK)
ROWS_PER_W = CHUNK * CHUNKS_PER_W


def _argmin_body(z_ref, cb_ref, idx_ref, idxf_ref, part_ref):
    z = z_ref[...]
    cb = cb_ref[...]
    dot2 = lax.dot_general(z, cb, (((1,), (1,)), ((), ())),
                           preferred_element_type=jnp.float32)
    z2 = jnp.sum(z * z, axis=1, keepdims=True) * 0.25
    c2 = jnp.sum(cb * cb, axis=1)
    d2 = (z2 + c2[None, :]) - dot2
    bm = jnp.min(d2, axis=1, keepdims=True)
    ii = lax.broadcasted_iota(jnp.int32, d2.shape, 1)
    loc = jnp.min(jnp.where(d2 == bm, ii, jnp.int32(K)), axis=1,
                  keepdims=True)
    idx_ref[...] = loc
    idxf_ref[...] = loc.astype(jnp.float32)
    part_ref[...] = jnp.broadcast_to(jnp.sum(jnp.maximum(bm, 0.0)),
                                     (1, 1, 128))


_distance_argmin = pl.pallas_call(
    _argmin_body,
    grid=(NB,),
    in_specs=[
        pl.BlockSpec((BN, D), lambda i: (i, 0)),
        pl.BlockSpec((K, D), lambda i: (0, 0)),
    ],
    out_specs=[
        pl.BlockSpec((BN, 1), lambda i: (i, 0)),
        pl.BlockSpec((BN, 1), lambda i: (i, 0)),
        pl.BlockSpec((1, 1, 128), lambda i: (i, 0, 0)),
    ],
    out_shape=[
        jax.ShapeDtypeStruct((N, 1), jnp.int32),
        jax.ShapeDtypeStruct((N, 1), jnp.float32),
        jax.ShapeDtypeStruct((NB, 1, 128), jnp.float32),
    ],
)


DPAD = 128


@functools.cache
def _make_sc_gather():
    @functools.partial(
        pl.kernel,
        out_type=jax.ShapeDtypeStruct((N, DPAD), jnp.float32),
        mesh=plsc.VectorSubcoreMesh(core_axis_name="c", subcore_axis_name="s"),
        scratch_types=[
            pltpu.VMEM((CHUNKS_PER_W, CHUNK), jnp.int32),
            pltpu.VMEM((CHUNK, DPAD), jnp.float32),
            pltpu.SemaphoreType.DMA,
        ],
    )
    def _sc_gather(idx_hbm, table_hbm, out_hbm, idx_v, rows_v, sem):
        wid = lax.axis_index("s") * NC + lax.axis_index("c")
        pltpu.sync_copy(
            idx_hbm.at[pl.ds(wid * CHUNKS_PER_W, CHUNKS_PER_W)], idx_v)
        base = wid * ROWS_PER_W
        for c in range(CHUNKS_PER_W):
            pltpu.async_copy(table_hbm.at[idx_v.at[c]], rows_v, sem).wait()
            pltpu.sync_copy(
                rows_v, out_hbm.at[pl.ds(base + c * CHUNK, CHUNK)])

    return _sc_gather


def kernel(z_feats, codebook):
    idx_i32, idx_f, parts = _distance_argmin(z_feats + z_feats, codebook)
    loss = jnp.sum(parts[:, 0, 0]) / jnp.float32(N * D)
    idx2d = idx_i32.reshape(NW * CHUNKS_PER_W, CHUNK)
    cb_pad = jnp.pad(codebook, ((0, 0), (0, DPAD - D)))
    quantized = _make_sc_gather()(idx2d, cb_pad)[:, :D]
    return quantized, loss, loss, idx_f


# speedup vs baseline: 1.7512x; 1.0240x over previous
import functools

import jax
import jax.numpy as jnp
from jax import lax
from jax.experimental import pallas as pl
from jax.experimental.pallas import tpu as pltpu
from jax.experimental.pallas import tpu_sc as plsc

N = 65536
D = 64
K = 8192
BN = 1024
NB = N // BN

NC = 2
NS = 16
NW = NC * NS
CHUNK = 128
CHUNKS_PER_W = N // (NW * CHUNK)
ROWS_PER_W = CHUNK * CHUNKS_PER_W


def _argmin_body(z_ref, cb_ref, idx_ref, idxf_ref, part_ref):
    z = z_ref[...]
    cb = cb_ref[...]
    dot2 = lax.dot_general(z + z, cb, (((1,), (1,)), ((), ())),
                           preferred_element_type=jnp.float32)
    z2 = jnp.sum(z * z, axis=1, keepdims=True)
    c2 = jnp.sum(cb * cb, axis=1)
    d2 = (z2 + c2[None, :]) - dot2
    bm = jnp.min(d2, axis=1, keepdims=True)
    ii = lax.broadcasted_iota(jnp.int32, d2.shape, 1)
    loc = jnp.min(jnp.where(d2 == bm, ii, jnp.int32(K)), axis=1,
                  keepdims=True)
    idx_ref[...] = loc
    idxf_ref[...] = loc.astype(jnp.float32)
    part_ref[...] = jnp.broadcast_to(jnp.sum(jnp.maximum(bm, 0.0)),
                                     (1, 1, 128))


_distance_argmin = pl.pallas_call(
    _argmin_body,
    grid=(NB,),
    in_specs=[
        pl.BlockSpec((BN, D), lambda i: (i, 0)),
        pl.BlockSpec((K, D), lambda i: (0, 0)),
    ],
    out_specs=[
        pl.BlockSpec((BN, 1), lambda i: (i, 0)),
        pl.BlockSpec((BN, 1), lambda i: (i, 0)),
        pl.BlockSpec((1, 1, 128), lambda i: (i, 0, 0)),
    ],
    out_shape=[
        jax.ShapeDtypeStruct((N, 1), jnp.int32),
        jax.ShapeDtypeStruct((N, 1), jnp.float32),
        jax.ShapeDtypeStruct((NB, 1, 128), jnp.float32),
    ],
)


DPAD = 128


@functools.cache
def _make_sc_gather():
    @functools.partial(
        pl.kernel,
        out_type=jax.ShapeDtypeStruct((N, DPAD), jnp.float32),
        mesh=plsc.VectorSubcoreMesh(core_axis_name="c", subcore_axis_name="s"),
        scratch_types=[
            pltpu.VMEM((CHUNKS_PER_W, CHUNK), jnp.int32),
            pltpu.VMEM((CHUNK, DPAD), jnp.float32),
            pltpu.SemaphoreType.DMA,
        ],
    )
    def _sc_gather(idx_hbm, table_hbm, out_hbm, idx_v, rows_v, sem):
        wid = lax.axis_index("s") * NC + lax.axis_index("c")
        pltpu.sync_copy(
            idx_hbm.at[pl.ds(wid * CHUNKS_PER_W, CHUNKS_PER_W)], idx_v)
        base = wid * ROWS_PER_W
        for c in range(CHUNKS_PER_W):
            pltpu.async_copy(table_hbm.at[idx_v.at[c]], rows_v, sem).wait()
            pltpu.sync_copy(
                rows_v, out_hbm.at[pl.ds(base + c * CHUNK, CHUNK)])

    return _sc_gather


def kernel(z_feats, codebook):
    idx_i32, idx_f, parts = _distance_argmin(z_feats, codebook)
    loss = jnp.sum(parts[:, 0, 0]) / jnp.float32(N * D)
    idx2d = idx_i32.reshape(NW * CHUNKS_PER_W, CHUNK)
    cb_pad = jnp.pad(codebook, ((0, 0), (0, DPAD - D)))
    quantized = _make_sc_gather()(idx2d, cb_pad)[:, :D]
    return quantized, loss, loss, idx_f
